# Initial kernel scaffold; baseline (speedup 1.0000x reference)
import jax
import jax.numpy as jnp
from jax.experimental import pallas as pl


def kernel(x0, ln0_scale, ln0_bias, Wr, br, W1, b1, ln1_scale, ln1_bias, W2, b2):
    raise NotImplementedError("write your pallas kernel here")


# speedup vs baseline: 1.2272x; 1.2272x over previous
import functools
import math

import jax
import jax.numpy as jnp
from jax import lax
from jax.experimental import pallas as pl
from jax.experimental.pallas import tpu as pltpu
from jax.experimental.pallas import tpu_sc as plsc

N = 2048
D = 768
H = 768
E = 8
C = 512
SENT = E * C
ROWS = E * C + 8

BLK = 256
NB = N // BLK

NC, NS = 2, 16
NW = NC * NS
TPW = N // NW


def _k1_body(x0_ref, s_ref, b_ref, wr_ref, br_ref,
             xact_ref, d0_ref, d1_ref, g0_ref, g1_ref, carry_ref):
    blk = pl.program_id(0)
    x = x0_ref[...]
    mean = jnp.mean(x, axis=-1, keepdims=True)
    var = jnp.mean((x - mean) ** 2, axis=-1, keepdims=True)
    xn = (x - mean) * lax.rsqrt(var + 1e-6) * s_ref[...] + b_ref[...]
    xa = jnp.maximum(xn, 0.0)
    xact_ref[...] = xa

    logits = jnp.dot(xa, wr_ref[...], preferred_element_type=jnp.float32)
    logits = logits + br_ref[...]

    eidx = lax.broadcasted_iota(jnp.int32, (BLK, E), 1)
    m0 = jnp.max(logits, axis=-1, keepdims=True)
    i0 = jnp.min(jnp.where(logits == m0, eidx, E), axis=-1, keepdims=True)
    neg = jnp.where(eidx == i0, -jnp.inf, logits)
    m1 = jnp.max(neg, axis=-1, keepdims=True)
    i1 = jnp.min(jnp.where(neg == m1, eidx, E), axis=-1, keepdims=True)

    e1 = jnp.exp(m1 - m0)
    g0_ref[...] = 1.0 / (1.0 + e1)
    g1_ref[...] = e1 / (1.0 + e1)

    oh0 = (eidx == i0).astype(jnp.float32)
    oh1 = (eidx == i1).astype(jnp.float32)
    ohs = oh0 + oh1

    r = lax.broadcasted_iota(jnp.int32, (BLK, BLK), 0)
    c = lax.broadcasted_iota(jnp.int32, (BLK, BLK), 1)
    tri = (c < r).astype(jnp.float32)
    cumb = jnp.dot(tri, ohs, preferred_element_type=jnp.float32)

    @pl.when(blk == 0)
    def _():
        carry_ref[...] = jnp.zeros((1, E), jnp.float32)

    base = cumb + carry_ref[...]
    carry_ref[...] = carry_ref[...] + jnp.sum(ohs, axis=0, keepdims=True)

    p0 = jnp.sum(base * oh0, axis=-1, keepdims=True).astype(jnp.int32)
    p1 = jnp.sum(base * oh1, axis=-1, keepdims=True).astype(jnp.int32)
    d0_ref[...] = jnp.where(p0 < C, i0 * C + p0, SENT)
    d1_ref[...] = jnp.where(p1 < C, i1 * C + p1, SENT)


def _k1_call(x0f, ln0_scale, ln0_bias, Wr, br):
    f32 = jnp.float32
    return pl.pallas_call(
        _k1_body,
        grid=(NB,),
        in_specs=[
            pl.BlockSpec((BLK, D), lambda b: (b, 0)),
            pl.BlockSpec((1, D), lambda b: (0, 0)),
            pl.BlockSpec((1, D), lambda b: (0, 0)),
            pl.BlockSpec((D, E), lambda b: (0, 0)),
            pl.BlockSpec((1, E), lambda b: (0, 0)),
        ],
        out_specs=[
            pl.BlockSpec((BLK, D), lambda b: (b, 0)),
            pl.BlockSpec((BLK, 1), lambda b: (b, 0)),
            pl.BlockSpec((BLK, 1), lambda b: (b, 0)),
            pl.BlockSpec((BLK, 1), lambda b: (b, 0)),
            pl.BlockSpec((BLK, 1), lambda b: (b, 0)),
        ],
        out_shape=[
            jax.ShapeDtypeStruct((N, D), f32),
            jax.ShapeDtypeStruct((N, 1), jnp.int32),
            jax.ShapeDtypeStruct((N, 1), jnp.int32),
            jax.ShapeDtypeStruct((N, 1), f32),
            jax.ShapeDtypeStruct((N, 1), f32),
        ],
        scratch_shapes=[pltpu.VMEM((1, E), f32)],
    )(x0f, ln0_scale.reshape(1, D), ln0_bias.reshape(1, D),
      Wr, br.reshape(1, E))


def _k2_body(x_hbm, d0_hbm, d1_hbm, xe_hbm, idx0_v, idx1_v, rows_v, sem):
    wid = lax.axis_index("s") * NC + lax.axis_index("c")
    base = wid * TPW
    pltpu.sync_copy(d0_hbm.at[pl.ds(base, TPW)], idx0_v)
    pltpu.sync_copy(d1_hbm.at[pl.ds(base, TPW)], idx1_v)
    pltpu.sync_copy(x_hbm.at[pl.ds(base, TPW)], rows_v)
    pltpu.async_copy(rows_v, xe_hbm.at[idx0_v], sem).wait()
    pltpu.async_copy(rows_v, xe_hbm.at[idx1_v], sem).wait()


@functools.cache
def _k2_kernel():
    return pl.kernel(
        _k2_body,
        mesh=plsc.VectorSubcoreMesh(core_axis_name="c", subcore_axis_name="s"),
        out_type=jax.ShapeDtypeStruct((ROWS, D), jnp.float32),
        scratch_types=[
            pltpu.VMEM((TPW,), jnp.int32),
            pltpu.VMEM((TPW,), jnp.int32),
            pltpu.VMEM((TPW, D), jnp.float32),
            pltpu.SemaphoreType.DMA,
        ],
    )


def _k2_call(xact, d0f, d1f):
    return _k2_kernel()(xact, d0f, d1f)


def _k3_body(xe_ref, w1_ref, b1_ref, s1_ref, bb1_ref, w2_ref, b2_ref, y_ref):
    x = xe_ref[...]
    h = jnp.dot(x, w1_ref[0], preferred_element_type=jnp.float32)
    h = h + b1_ref[0]
    mean = jnp.mean(h, axis=-1, keepdims=True)
    var = jnp.mean((h - mean) ** 2, axis=-1, keepdims=True)
    h = (h - mean) * lax.rsqrt(var + 1e-6) * s1_ref[0] + bb1_ref[0]
    h = jnp.maximum(h, 0.0)
    y = jnp.dot(h, w2_ref[0], preferred_element_type=jnp.float32)
    y_ref[...] = y + b2_ref[0]


def _k3_call(xe, W1, b1, ln1_scale, ln1_bias, W2, b2):
    vec = pl.BlockSpec((1, 1, H), lambda e: (e, 0, 0))
    return pl.pallas_call(
        _k3_body,
        grid=(E,),
        in_specs=[
            pl.BlockSpec((C, D), lambda e: (e, 0)),
            pl.BlockSpec((1, D, H), lambda e: (e, 0, 0)),
            vec, vec, vec,
            pl.BlockSpec((1, H, D), lambda e: (e, 0, 0)),
            pl.BlockSpec((1, 1, D), lambda e: (e, 0, 0)),
        ],
        out_specs=pl.BlockSpec((C, D), lambda e: (e, 0)),
        out_shape=jax.ShapeDtypeStruct((ROWS, D), jnp.float32),
    )(xe, W1, b1.reshape(E, 1, H), ln1_scale.reshape(E, 1, H),
      ln1_bias.reshape(E, 1, H), W2, b2.reshape(E, 1, D))


def _k4_body(y_hbm, d0_hbm, d1_hbm, yc0_hbm, yc1_hbm,
             idx0_v, idx1_v, r0_v, r1_v, sem):
    wid = lax.axis_index("s") * NC + lax.axis_index("c")
    base = wid * TPW
    pltpu.sync_copy(d0_hbm.at[pl.ds(base, TPW)], idx0_v)
    pltpu.sync_copy(d1_hbm.at[pl.ds(base, TPW)], idx1_v)
    pltpu.async_copy(y_hbm.at[idx0_v], r0_v, sem).wait()
    pltpu.async_copy(y_hbm.at[idx1_v], r1_v, sem).wait()
    pltpu.sync_copy(r0_v, yc0_hbm.at[pl.ds(base, TPW)])
    pltpu.sync_copy(r1_v, yc1_hbm.at[pl.ds(base, TPW)])


@functools.cache
def _k4_kernel():
    return pl.kernel(
        _k4_body,
        mesh=plsc.VectorSubcoreMesh(core_axis_name="c", subcore_axis_name="s"),
        out_type=[jax.ShapeDtypeStruct((N, D), jnp.float32),
                  jax.ShapeDtypeStruct((N, D), jnp.float32)],
        scratch_types=[
            pltpu.VMEM((TPW,), jnp.int32),
            pltpu.VMEM((TPW,), jnp.int32),
            pltpu.VMEM((TPW, D), jnp.float32),
            pltpu.VMEM((TPW, D), jnp.float32),
            pltpu.SemaphoreType.DMA,
        ],
    )


def _k4_call(ybuf, d0f, d1f):
    return _k4_kernel()(ybuf, d0f, d1f)


def _k5_body(x0_ref, y0_ref, y1_ref, g0_ref, g1_ref, d0_ref, d1_ref, o_ref):
    c0 = jnp.where(d0_ref[...] < SENT, g0_ref[...] * y0_ref[...], 0.0)
    c1 = jnp.where(d1_ref[...] < SENT, g1_ref[...] * y1_ref[...], 0.0)
    o_ref[...] = x0_ref[...] + c0 + c1


def _k5_call(x0f, yc0, yc1, g0, g1, d0, d1):
    row = pl.BlockSpec((BLK, D), lambda b: (b, 0))
    col = pl.BlockSpec((BLK, 1), lambda b: (b, 0))
    return pl.pallas_call(
        _k5_body,
        grid=(NB,),
        in_specs=[row, row, row, col, col, col, col],
        out_specs=row,
        out_shape=jax.ShapeDtypeStruct((N, D), jnp.float32),
    )(x0f, yc0, yc1, g0, g1, d0, d1)


def kernel(x0, ln0_scale, ln0_bias, Wr, br, W1, b1, ln1_scale, ln1_bias, W2, b2):
    x0f = x0.reshape(N, D)
    xact, d0, d1, g0, g1 = _k1_call(x0f, ln0_scale, ln0_bias, Wr, br)
    d0f = d0.reshape(N)
    d1f = d1.reshape(N)
    xe = _k2_call(xact, d0f, d1f)
    ybuf = _k3_call(xe, W1, b1, ln1_scale, ln1_bias, W2, b2)
    yc0, yc1 = _k4_call(ybuf, d0f, d1f)
    out = _k5_call(x0f, yc0, yc1, g0, g1, d0, d1)
    return out.reshape(x0.shape)
